# baseline (speedup 1.0000x reference)
import functools

import jax
import jax.numpy as jnp
import numpy as np
from jax import lax
from jax.experimental import pallas as pl
from jax.experimental.pallas import tpu as pltpu
from jax.experimental.pallas import tpu_sc as plsc

_POS_DIM = 3
_DP1 = _POS_DIM + 1
_N_LEVELS = 16
_N_FEATS = 2
_HASHMAP_SIZE = 2 ** 19
_N_POINTS = 524288
_RES_LIST = np.array([16.0, 22.11, 30.56, 42.23, 58.35, 80.65, 111.45, 154.01,
                      212.84, 294.13, 406.47, 561.71, 776.23, 1072.69, 1482.44,
                      2048.0], dtype=np.float64)
_PRIMES = (1, 2654435761, 805459861)
_inv = 1.0 / np.sqrt((np.arange(_POS_DIM) + 1.0) * (np.arange(_POS_DIM) + 2.0))
_SCALE_NP = np.asarray(_RES_LIST[:, None] * _inv[None, :], dtype=np.float32)

_NC = 2
_NS = 16
_NW = _NC * _NS
_LANES = 16
_PTS_W = _N_POINTS // _NW
_CHUNK = 1024
_NCHUNK = _PTS_W // _CHUNK
_NV = _CHUNK // _LANES
_GSUB = 128
_NG = _DP1 * _CHUNK // _GSUB


_GATHER_DN = lax.GatherDimensionNumbers(
    offset_dims=(), collapsed_slice_dims=(0,), start_index_map=(0,))


def _lane_gather(v, idx):
    return lax.gather(v, idx[:, None], _GATHER_DN, (1,),
                      mode=lax.GatherScatterMode.PROMISE_IN_BOUNDS)


def _floor_f32(v):
    t = v.astype(jnp.int32).astype(jnp.float32)
    return t - jnp.where(v < t, jnp.float32(1.0), jnp.float32(0.0))


def _ceil_f32(v):
    t = v.astype(jnp.int32).astype(jnp.float32)
    return t + jnp.where(v > t, jnp.float32(1.0), jnp.float32(0.0))


def _vertex_data(x, y, z, sc0, sc1, sc2, sh0, sh1, sh2):
    p0 = (x + sh0) * sc0
    p1 = (y + sh1) * sc1
    p2 = (z + sh2) * sc2
    s1 = p2 + p1
    e0 = s1 + p0
    e1 = s1 - p0
    e2 = p2 - 2.0 * p1
    e3 = -(3.0 * p2)
    elev = (e0, e1, e2, e3)
    rem = []
    remf = []
    for e in elev:
        v = e / 4.0
        up = _ceil_f32(v) * 4.0
        down = _floor_f32(v) * 4.0
        rf = jnp.where(up - e < e - down, up, down)
        remf.append(rf)
        rem.append(rf.astype(jnp.int32))
    ssum = (rem[0] + rem[1]) + (rem[2] + rem[3])
    s_vec = lax.shift_right_arithmetic(ssum, 2)
    one_i = jnp.full((_LANES,), 1, jnp.int32)
    zero_i = jnp.zeros((_LANES,), jnp.int32)
    d = [e - rf for e, rf in zip(elev, remf)]
    rank = [s_vec, s_vec, s_vec, s_vec]
    for i in range(_DP1):
        for j in range(i + 1, _DP1):
            c = jnp.where(d[j] > d[i], one_i, zero_i)
            rank[i] = rank[i] + c
            rank[j] = rank[j] + (one_i - c)
    for i in range(_DP1):
        adj = (jnp.where(rank[i] < 0, 4, 0) - jnp.where(rank[i] > _POS_DIM, 4, 0)).astype(jnp.int32)
        rank[i] = rank[i] + adj
        rem[i] = rem[i] + adj
    zero_f = jnp.zeros((_LANES,), jnp.float32)
    delta = [(elev[i] - rem[i].astype(jnp.float32)) / 4.0 for i in range(_DP1)]
    g = []
    for j in range(_DP1):
        acc = zero_f
        for i in range(_DP1):
            acc = acc + jnp.where(rank[i] == j, delta[i], zero_f)
        g.append(acc)
    bary0 = g[3] + (jnp.float32(1.0) - g[0])
    bary1 = g[2] - g[3]
    bary2 = g[1] - g[2]
    bary3 = g[0] - g[1]
    u0 = rem[0].astype(jnp.uint32)
    m1 = rem[1].astype(jnp.uint32) * jnp.uint32(_PRIMES[1])
    m2 = rem[2].astype(jnp.uint32) * jnp.uint32(_PRIMES[2])
    mask = jnp.uint32(_HASHMAP_SIZE - 1)
    idxs = []
    for r in range(_DP1):
        if r == 0:
            t0, t1, t2 = u0, m1, m2
        else:
            cr = jnp.uint32((r * _PRIMES[0]) & 0xFFFFFFFF)
            crm4 = jnp.uint32(((r - 4) * _PRIMES[0]) & 0xFFFFFFFF)
            t0 = u0 + jnp.where(rank[0] > _POS_DIM - r, crm4, cr)
            cr = jnp.uint32((r * _PRIMES[1]) & 0xFFFFFFFF)
            crm4 = jnp.uint32(((r - 4) * _PRIMES[1]) & 0xFFFFFFFF)
            t1 = m1 + jnp.where(rank[1] > _POS_DIM - r, crm4, cr)
            cr = jnp.uint32((r * _PRIMES[2]) & 0xFFFFFFFF)
            crm4 = jnp.uint32(((r - 4) * _PRIMES[2]) & 0xFFFFFFFF)
            t2 = m2 + jnp.where(rank[2] > _POS_DIM - r, crm4, cr)
        h = (t0 ^ t1) ^ t2
        idxs.append((h & mask).astype(jnp.int32))
    return idxs, (bary0, bary1, bary2, bary3)


def _sc_body(px_hbm, py_hbm, pz_hbm, tab_hbm, cst_hbm, out_hbm,
             xs, ys, zs, cst_v, idx0_v, idx1_v, bary_v, rows0_v, rows1_v,
             ob_v, sem0, sem1):
    c = lax.axis_index("c")
    s = lax.axis_index("s")
    wid = s * _NC + c
    base = wid * _PTS_W
    pltpu.sync_copy(px_hbm.at[pl.ds(base, _PTS_W)], xs)
    pltpu.sync_copy(py_hbm.at[pl.ds(base, _PTS_W)], ys)
    pltpu.sync_copy(pz_hbm.at[pl.ds(base, _PTS_W)], zs)
    pltpu.sync_copy(cst_hbm, cst_v)
    sems = (sem0, sem1)
    iota16 = lax.iota(jnp.int32, _LANES)
    _NF = _N_LEVELS * _N_FEATS

    @pl.loop(0, _NCHUNK)
    def _chunk(ch):
        off = ch * _CHUNK

        def _compute(l):
            so = (l & 1) * (_DP1 * _CHUNK)
            cb = l * (6 * _LANES)
            sc0 = cst_v[pl.ds(cb + 0 * _LANES, _LANES)]
            sc1 = cst_v[pl.ds(cb + 1 * _LANES, _LANES)]
            sc2 = cst_v[pl.ds(cb + 2 * _LANES, _LANES)]
            sh0 = cst_v[pl.ds(cb + 3 * _LANES, _LANES)]
            sh1 = cst_v[pl.ds(cb + 4 * _LANES, _LANES)]
            sh2 = cst_v[pl.ds(cb + 5 * _LANES, _LANES)]
            loff = jnp.full((_LANES,), l * _HASHMAP_SIZE, jnp.int32)

            @pl.loop(0, _NV)
            def _vcompute(v):
                po = off + v * _LANES
                x = xs[pl.ds(po, _LANES)]
                y = ys[pl.ds(po, _LANES)]
                z = zs[pl.ds(po, _LANES)]
                idxs, barys = _vertex_data(x, y, z, sc0, sc1, sc2, sh0, sh1, sh2)
                for r in range(_DP1):
                    e = (idxs[r] + loff) * 2
                    idx0_v[pl.ds(so + r * _CHUNK + v * _LANES, _LANES)] = e
                    idx1_v[pl.ds(so + r * _CHUNK + v * _LANES, _LANES)] = e + 1
                    bary_v[pl.ds(so + r * _CHUNK + v * _LANES, _LANES)] = barys[r]

        def _fire(l):
            so = (l & 1) * (_DP1 * _CHUNK)
            sem = sems[l & 1]

            @pl.loop(0, _NG)
            def _gfire(g):
                pltpu.async_copy(tab_hbm.at[idx0_v.at[pl.ds(so + g * _GSUB, _GSUB)]],
                                 rows0_v.at[pl.ds(so + g * _GSUB, _GSUB)], sem)
                pltpu.async_copy(tab_hbm.at[idx1_v.at[pl.ds(so + g * _GSUB, _GSUB)]],
                                 rows1_v.at[pl.ds(so + g * _GSUB, _GSUB)], sem)

        def _drain_blend(l):
            so = (l & 1) * (_DP1 * _CHUNK)
            sem = sems[l & 1]
            pltpu.make_async_copy(tab_hbm.at[pl.ds(0, _DP1 * _CHUNK)],
                                  rows0_v.at[pl.ds(so, _DP1 * _CHUNK)], sem).wait()
            pltpu.make_async_copy(tab_hbm.at[pl.ds(0, _DP1 * _CHUNK)],
                                  rows1_v.at[pl.ds(so, _DP1 * _CHUNK)], sem).wait()

            @pl.loop(0, _NV)
            def _vblend(v):
                acc0 = jnp.zeros((_LANES,), jnp.float32)
                acc1 = jnp.zeros((_LANES,), jnp.float32)
                for r in range(_DP1):
                    q = so + r * _CHUNK + v * _LANES
                    b = bary_v[pl.ds(q, _LANES)]
                    acc0 = acc0 + b * rows0_v[pl.ds(q, _LANES)]
                    acc1 = acc1 + b * rows1_v[pl.ds(q, _LANES)]
                ob_v[pl.ds(v * _LANES, _LANES)] = acc0
                ob_v[pl.ds(_CHUNK + v * _LANES, _LANES)] = acc1

            obase = (2 * l) * _N_POINTS + base + off
            pltpu.sync_copy(ob_v.at[pl.ds(0, _CHUNK)], out_hbm.at[pl.ds(obase, _CHUNK)])
            pltpu.sync_copy(ob_v.at[pl.ds(_CHUNK, _CHUNK)],
                            out_hbm.at[pl.ds(obase + _N_POINTS, _CHUNK)])

        _compute(0)
        _fire(0)
        for l in range(1, _N_LEVELS):
            _compute(l)
            _fire(l)
            _drain_blend(l - 1)
        _drain_blend(_N_LEVELS - 1)


@jax.jit
def _encode_sc(px, py, pz, tab, cst):
    mesh = plsc.VectorSubcoreMesh(core_axis_name="c", subcore_axis_name="s")
    return pl.kernel(
        _sc_body,
        out_type=jax.ShapeDtypeStruct((_N_LEVELS * _N_FEATS * _N_POINTS,), jnp.float32),
        mesh=mesh,
        scratch_types=[
            pltpu.VMEM((_PTS_W,), jnp.float32),
            pltpu.VMEM((_PTS_W,), jnp.float32),
            pltpu.VMEM((_PTS_W,), jnp.float32),
            pltpu.VMEM((_N_LEVELS * 6 * _LANES,), jnp.float32),
            pltpu.VMEM((2 * _DP1 * _CHUNK,), jnp.int32),
            pltpu.VMEM((2 * _DP1 * _CHUNK,), jnp.int32),
            pltpu.VMEM((2 * _DP1 * _CHUNK,), jnp.float32),
            pltpu.VMEM((2 * _DP1 * _CHUNK,), jnp.float32),
            pltpu.VMEM((2 * _DP1 * _CHUNK,), jnp.float32),
            pltpu.VMEM((2 * _CHUNK,), jnp.float32),
            pltpu.SemaphoreType.DMA,
            pltpu.SemaphoreType.DMA,
        ],
    )(px, py, pz, tab, cst)


_TBLK = 2048
_PBLK = 2048


def _transpose_body(x_ref, o_ref):
    o_ref[...] = x_ref[...].T


def _prep_body(p_ref, s_ref, x_ref, y_ref, z_ref):
    sc = p_ref[...] * s_ref[...]
    x_ref[...] = sc[:, 0]
    y_ref[...] = sc[:, 1]
    z_ref[...] = sc[:, 2]


@jax.jit
def _prep_tc(positions, pos_scale):
    vec = jax.ShapeDtypeStruct((_N_POINTS,), jnp.float32)
    return pl.pallas_call(
        _prep_body,
        grid=(_N_POINTS // _PBLK,),
        in_specs=[pl.BlockSpec((_PBLK, _POS_DIM), lambda i: (i, 0)),
                  pl.BlockSpec((1, _POS_DIM), lambda i: (0, 0))],
        out_specs=[pl.BlockSpec((_PBLK,), lambda i: (i,))] * 3,
        out_shape=[vec, vec, vec],
    )(positions, pos_scale[None, :])


@jax.jit
def _transpose_tc(y):
    nf = _N_LEVELS * _N_FEATS
    return pl.pallas_call(
        _transpose_body,
        grid=(_N_POINTS // _TBLK,),
        in_specs=[pl.BlockSpec((nf, _TBLK), lambda i: (0, i))],
        out_specs=pl.BlockSpec((_TBLK, nf), lambda i: (i, 0)),
        out_shape=jax.ShapeDtypeStruct((_N_POINTS, nf), jnp.float32),
    )(y)


def kernel(positions, lattice_values, level_random_shifts, pos_scale):
    px, py, pz = _prep_tc(positions, pos_scale)
    tab = lattice_values.reshape(_N_LEVELS * _HASHMAP_SIZE * _N_FEATS)
    cst = jnp.zeros((_N_LEVELS, 6), jnp.float32)
    cst = cst.at[:, 0:3].set(jnp.asarray(_SCALE_NP))
    cst = cst.at[:, 3:6].set(level_random_shifts.astype(jnp.float32))
    cst = jnp.broadcast_to(cst[:, :, None], (_N_LEVELS, 6, _LANES)).reshape(-1)
    out = _encode_sc(px, py, pz, tab, cst)
    return _transpose_tc(out.reshape(_N_LEVELS * _N_FEATS, _N_POINTS))


# speedup vs baseline: 1.4162x; 1.4162x over previous
import functools

import jax
import jax.numpy as jnp
import numpy as np
from jax import lax
from jax.experimental import pallas as pl
from jax.experimental.pallas import tpu as pltpu
from jax.experimental.pallas import tpu_sc as plsc

_POS_DIM = 3
_DP1 = _POS_DIM + 1
_N_LEVELS = 16
_N_FEATS = 2
_HASHMAP_SIZE = 2 ** 19
_N_POINTS = 524288
_RES_LIST = np.array([16.0, 22.11, 30.56, 42.23, 58.35, 80.65, 111.45, 154.01,
                      212.84, 294.13, 406.47, 561.71, 776.23, 1072.69, 1482.44,
                      2048.0], dtype=np.float64)
_PRIMES = (1, 2654435761, 805459861)
_inv = 1.0 / np.sqrt((np.arange(_POS_DIM) + 1.0) * (np.arange(_POS_DIM) + 2.0))
_SCALE_NP = np.asarray(_RES_LIST[:, None] * _inv[None, :], dtype=np.float32)

_NC = 2
_NS = 16
_NW = _NC * _NS
_LANES = 16
_PTS_W = _N_POINTS // _NW
_CHUNK = 1024
_NCHUNK = _PTS_W // _CHUNK
_NV = _CHUNK // _LANES
_GSUB = 128
_NG = _DP1 * _CHUNK // _GSUB


_GATHER_DN = lax.GatherDimensionNumbers(
    offset_dims=(), collapsed_slice_dims=(0,), start_index_map=(0,))


def _lane_gather(v, idx):
    return lax.gather(v, idx[:, None], _GATHER_DN, (1,),
                      mode=lax.GatherScatterMode.PROMISE_IN_BOUNDS)


def _floor_f32(v):
    t = v.astype(jnp.int32).astype(jnp.float32)
    return t - jnp.where(v < t, jnp.float32(1.0), jnp.float32(0.0))


def _ceil_f32(v):
    t = v.astype(jnp.int32).astype(jnp.float32)
    return t + jnp.where(v > t, jnp.float32(1.0), jnp.float32(0.0))


def _vertex_data(x, y, z, sc0, sc1, sc2, sh0, sh1, sh2):
    p0 = (x + sh0) * sc0
    p1 = (y + sh1) * sc1
    p2 = (z + sh2) * sc2
    s1 = p2 + p1
    e0 = s1 + p0
    e1 = s1 - p0
    e2 = p2 - 2.0 * p1
    e3 = -(3.0 * p2)
    elev = (e0, e1, e2, e3)
    rem = []
    remf = []
    for e in elev:
        v = e / 4.0
        up = _ceil_f32(v) * 4.0
        down = _floor_f32(v) * 4.0
        rf = jnp.where(up - e < e - down, up, down)
        remf.append(rf)
        rem.append(rf.astype(jnp.int32))
    ssum = (rem[0] + rem[1]) + (rem[2] + rem[3])
    s_vec = lax.shift_right_arithmetic(ssum, 2)
    one_i = jnp.full((_LANES,), 1, jnp.int32)
    zero_i = jnp.zeros((_LANES,), jnp.int32)
    d = [e - rf for e, rf in zip(elev, remf)]
    rank = [s_vec, s_vec, s_vec, s_vec]
    for i in range(_DP1):
        for j in range(i + 1, _DP1):
            c = jnp.where(d[j] > d[i], one_i, zero_i)
            rank[i] = rank[i] + c
            rank[j] = rank[j] + (one_i - c)
    for i in range(_DP1):
        adj = (jnp.where(rank[i] < 0, 4, 0) - jnp.where(rank[i] > _POS_DIM, 4, 0)).astype(jnp.int32)
        rank[i] = rank[i] + adj
        rem[i] = rem[i] + adj
    zero_f = jnp.zeros((_LANES,), jnp.float32)
    delta = [(elev[i] - rem[i].astype(jnp.float32)) / 4.0 for i in range(_DP1)]
    g = []
    for j in range(_DP1):
        acc = zero_f
        for i in range(_DP1):
            acc = acc + jnp.where(rank[i] == j, delta[i], zero_f)
        g.append(acc)
    bary0 = g[3] + (jnp.float32(1.0) - g[0])
    bary1 = g[2] - g[3]
    bary2 = g[1] - g[2]
    bary3 = g[0] - g[1]
    u0 = rem[0].astype(jnp.uint32)
    m1 = rem[1].astype(jnp.uint32) * jnp.uint32(_PRIMES[1])
    m2 = rem[2].astype(jnp.uint32) * jnp.uint32(_PRIMES[2])
    mask = jnp.uint32(_HASHMAP_SIZE - 1)
    idxs = []
    for r in range(_DP1):
        if r == 0:
            t0, t1, t2 = u0, m1, m2
        else:
            cr = jnp.uint32((r * _PRIMES[0]) & 0xFFFFFFFF)
            crm4 = jnp.uint32(((r - 4) * _PRIMES[0]) & 0xFFFFFFFF)
            t0 = u0 + jnp.where(rank[0] > _POS_DIM - r, crm4, cr)
            cr = jnp.uint32((r * _PRIMES[1]) & 0xFFFFFFFF)
            crm4 = jnp.uint32(((r - 4) * _PRIMES[1]) & 0xFFFFFFFF)
            t1 = m1 + jnp.where(rank[1] > _POS_DIM - r, crm4, cr)
            cr = jnp.uint32((r * _PRIMES[2]) & 0xFFFFFFFF)
            crm4 = jnp.uint32(((r - 4) * _PRIMES[2]) & 0xFFFFFFFF)
            t2 = m2 + jnp.where(rank[2] > _POS_DIM - r, crm4, cr)
        h = (t0 ^ t1) ^ t2
        idxs.append((h & mask).astype(jnp.int32))
    return idxs, (bary0, bary1, bary2, bary3)


def _sc_body(px_hbm, py_hbm, pz_hbm, tab0_hbm, tab1_hbm, cst_hbm, out_hbm,
             xs, ys, zs, cst_v, idx_v, bary_v, rows0_v, rows1_v,
             ob_v, sem0, sem1):
    c = lax.axis_index("c")
    s = lax.axis_index("s")
    wid = s * _NC + c
    base = wid * _PTS_W
    pltpu.sync_copy(px_hbm.at[pl.ds(base, _PTS_W)], xs)
    pltpu.sync_copy(py_hbm.at[pl.ds(base, _PTS_W)], ys)
    pltpu.sync_copy(pz_hbm.at[pl.ds(base, _PTS_W)], zs)
    pltpu.sync_copy(cst_hbm, cst_v)
    sems = (sem0, sem1)
    iota16 = lax.iota(jnp.int32, _LANES)
    _NF = _N_LEVELS * _N_FEATS

    @pl.loop(0, _NCHUNK)
    def _chunk(ch):
        off = ch * _CHUNK

        def _compute(l):
            so = (l & 1) * (_DP1 * _CHUNK)
            cb = l * (6 * _LANES)
            sc0 = cst_v[pl.ds(cb + 0 * _LANES, _LANES)]
            sc1 = cst_v[pl.ds(cb + 1 * _LANES, _LANES)]
            sc2 = cst_v[pl.ds(cb + 2 * _LANES, _LANES)]
            sh0 = cst_v[pl.ds(cb + 3 * _LANES, _LANES)]
            sh1 = cst_v[pl.ds(cb + 4 * _LANES, _LANES)]
            sh2 = cst_v[pl.ds(cb + 5 * _LANES, _LANES)]
            loff = jnp.full((_LANES,), l * _HASHMAP_SIZE, jnp.int32)

            @pl.loop(0, _NV)
            def _vcompute(v):
                po = off + v * _LANES
                x = xs[pl.ds(po, _LANES)]
                y = ys[pl.ds(po, _LANES)]
                z = zs[pl.ds(po, _LANES)]
                idxs, barys = _vertex_data(x, y, z, sc0, sc1, sc2, sh0, sh1, sh2)
                for r in range(_DP1):
                    idx_v[pl.ds(so + r * _CHUNK + v * _LANES, _LANES)] = idxs[r] + loff
                    bary_v[pl.ds(so + r * _CHUNK + v * _LANES, _LANES)] = barys[r]

        def _fire(l):
            so = (l & 1) * (_DP1 * _CHUNK)
            sem = sems[l & 1]

            @pl.loop(0, _NG)
            def _gfire(g):
                pltpu.async_copy(tab0_hbm.at[idx_v.at[pl.ds(so + g * _GSUB, _GSUB)]],
                                 rows0_v.at[pl.ds(so + g * _GSUB, _GSUB)], sem)
                pltpu.async_copy(tab1_hbm.at[idx_v.at[pl.ds(so + g * _GSUB, _GSUB)]],
                                 rows1_v.at[pl.ds(so + g * _GSUB, _GSUB)], sem)

        def _drain_blend(l):
            so = (l & 1) * (_DP1 * _CHUNK)
            sem = sems[l & 1]
            pltpu.make_async_copy(tab0_hbm.at[pl.ds(0, _DP1 * _CHUNK)],
                                  rows0_v.at[pl.ds(so, _DP1 * _CHUNK)], sem).wait()
            pltpu.make_async_copy(tab1_hbm.at[pl.ds(0, _DP1 * _CHUNK)],
                                  rows1_v.at[pl.ds(so, _DP1 * _CHUNK)], sem).wait()

            @pl.loop(0, _NV)
            def _vblend(v):
                acc0 = jnp.zeros((_LANES,), jnp.float32)
                acc1 = jnp.zeros((_LANES,), jnp.float32)
                for r in range(_DP1):
                    q = so + r * _CHUNK + v * _LANES
                    b = bary_v[pl.ds(q, _LANES)]
                    acc0 = acc0 + b * rows0_v[pl.ds(q, _LANES)]
                    acc1 = acc1 + b * rows1_v[pl.ds(q, _LANES)]
                ob_v[pl.ds(v * _LANES, _LANES)] = acc0
                ob_v[pl.ds(_CHUNK + v * _LANES, _LANES)] = acc1

            obase = (2 * l) * _N_POINTS + base + off
            pltpu.sync_copy(ob_v.at[pl.ds(0, _CHUNK)], out_hbm.at[pl.ds(obase, _CHUNK)])
            pltpu.sync_copy(ob_v.at[pl.ds(_CHUNK, _CHUNK)],
                            out_hbm.at[pl.ds(obase + _N_POINTS, _CHUNK)])

        _compute(0)
        _fire(0)
        for l in range(1, _N_LEVELS):
            _compute(l)
            _fire(l)
            _drain_blend(l - 1)
        _drain_blend(_N_LEVELS - 1)


@jax.jit
def _encode_sc(px, py, pz, tab0, tab1, cst):
    mesh = plsc.VectorSubcoreMesh(core_axis_name="c", subcore_axis_name="s")
    return pl.kernel(
        _sc_body,
        out_type=jax.ShapeDtypeStruct((_N_LEVELS * _N_FEATS * _N_POINTS,), jnp.float32),
        mesh=mesh,
        scratch_types=[
            pltpu.VMEM((_PTS_W,), jnp.float32),
            pltpu.VMEM((_PTS_W,), jnp.float32),
            pltpu.VMEM((_PTS_W,), jnp.float32),
            pltpu.VMEM((_N_LEVELS * 6 * _LANES,), jnp.float32),
            pltpu.VMEM((2 * _DP1 * _CHUNK,), jnp.int32),
            pltpu.VMEM((2 * _DP1 * _CHUNK,), jnp.float32),
            pltpu.VMEM((2 * _DP1 * _CHUNK,), jnp.float32),
            pltpu.VMEM((2 * _DP1 * _CHUNK,), jnp.float32),
            pltpu.VMEM((2 * _CHUNK,), jnp.float32),
            pltpu.SemaphoreType.DMA,
            pltpu.SemaphoreType.DMA,
        ],
    )(px, py, pz, tab0, tab1, cst)


_TBLK = 2048
_PBLK = 2048


def _transpose_body(x_ref, o_ref):
    o_ref[...] = x_ref[...].T


_FBLK = 8192


def _split_body(a_ref, o0_ref, o1_ref):
    v = a_ref[...]
    o0_ref[...] = v[:, 0]
    o1_ref[...] = v[:, 1]


@jax.jit
def _split_tc(lattice_values):
    nb = _HASHMAP_SIZE // _FBLK
    plane = jax.ShapeDtypeStruct((_N_LEVELS * _HASHMAP_SIZE,), jnp.float32)
    return pl.pallas_call(
        _split_body,
        grid=(_N_LEVELS * nb,),
        in_specs=[pl.BlockSpec((None, _FBLK, _N_FEATS), lambda i: (i // nb, i % nb, 0))],
        out_specs=[pl.BlockSpec((_FBLK,), lambda i: (i,)),
                   pl.BlockSpec((_FBLK,), lambda i: (i,))],
        out_shape=[plane, plane],
    )(lattice_values)


def _prep_body(p_ref, s_ref, x_ref, y_ref, z_ref):
    sc = p_ref[...] * s_ref[...]
    x_ref[...] = sc[:, 0]
    y_ref[...] = sc[:, 1]
    z_ref[...] = sc[:, 2]


@jax.jit
def _prep_tc(positions, pos_scale):
    vec = jax.ShapeDtypeStruct((_N_POINTS,), jnp.float32)
    return pl.pallas_call(
        _prep_body,
        grid=(_N_POINTS // _PBLK,),
        in_specs=[pl.BlockSpec((_PBLK, _POS_DIM), lambda i: (i, 0)),
                  pl.BlockSpec((1, _POS_DIM), lambda i: (0, 0))],
        out_specs=[pl.BlockSpec((_PBLK,), lambda i: (i,))] * 3,
        out_shape=[vec, vec, vec],
    )(positions, pos_scale[None, :])


@jax.jit
def _transpose_tc(y):
    nf = _N_LEVELS * _N_FEATS
    return pl.pallas_call(
        _transpose_body,
        grid=(_N_POINTS // _TBLK,),
        in_specs=[pl.BlockSpec((nf, _TBLK), lambda i: (0, i))],
        out_specs=pl.BlockSpec((_TBLK, nf), lambda i: (i, 0)),
        out_shape=jax.ShapeDtypeStruct((_N_POINTS, nf), jnp.float32),
    )(y)


def kernel(positions, lattice_values, level_random_shifts, pos_scale):
    px, py, pz = _prep_tc(positions, pos_scale)
    tab0, tab1 = _split_tc(lattice_values)
    cst = jnp.zeros((_N_LEVELS, 6), jnp.float32)
    cst = cst.at[:, 0:3].set(jnp.asarray(_SCALE_NP))
    cst = cst.at[:, 3:6].set(level_random_shifts.astype(jnp.float32))
    cst = jnp.broadcast_to(cst[:, :, None], (_N_LEVELS, 6, _LANES)).reshape(-1)
    out = _encode_sc(px, py, pz, tab0, tab1, cst)
    return _transpose_tc(out.reshape(_N_LEVELS * _N_FEATS, _N_POINTS))


# speedup vs baseline: 1.4222x; 1.0043x over previous
import functools

import jax
import jax.numpy as jnp
import numpy as np
from jax import lax
from jax.experimental import pallas as pl
from jax.experimental.pallas import tpu as pltpu
from jax.experimental.pallas import tpu_sc as plsc

_POS_DIM = 3
_DP1 = _POS_DIM + 1
_N_LEVELS = 16
_N_FEATS = 2
_HASHMAP_SIZE = 2 ** 19
_N_POINTS = 524288
_RES_LIST = np.array([16.0, 22.11, 30.56, 42.23, 58.35, 80.65, 111.45, 154.01,
                      212.84, 294.13, 406.47, 561.71, 776.23, 1072.69, 1482.44,
                      2048.0], dtype=np.float64)
_PRIMES = (1, 2654435761, 805459861)
_inv = 1.0 / np.sqrt((np.arange(_POS_DIM) + 1.0) * (np.arange(_POS_DIM) + 2.0))
_SCALE_NP = np.asarray(_RES_LIST[:, None] * _inv[None, :], dtype=np.float32)

_NC = 2
_NS = 16
_NW = _NC * _NS
_LANES = 16
_PTS_W = _N_POINTS // _NW
_CHUNK = 1024
_NCHUNK = _PTS_W // _CHUNK
_NV = _CHUNK // _LANES
_GSUB = 128
_NG = _DP1 * _CHUNK // _GSUB


_GATHER_DN = lax.GatherDimensionNumbers(
    offset_dims=(), collapsed_slice_dims=(0,), start_index_map=(0,))


def _lane_gather(v, idx):
    return lax.gather(v, idx[:, None], _GATHER_DN, (1,),
                      mode=lax.GatherScatterMode.PROMISE_IN_BOUNDS)


def _floor_f32(v):
    t = v.astype(jnp.int32).astype(jnp.float32)
    return t - jnp.where(v < t, jnp.float32(1.0), jnp.float32(0.0))


def _ceil_f32(v):
    t = v.astype(jnp.int32).astype(jnp.float32)
    return t + jnp.where(v > t, jnp.float32(1.0), jnp.float32(0.0))


def _vertex_data(x, y, z, sc0, sc1, sc2, sh0, sh1, sh2):
    p0 = (x + sh0) * sc0
    p1 = (y + sh1) * sc1
    p2 = (z + sh2) * sc2
    s1 = p2 + p1
    e0 = s1 + p0
    e1 = s1 - p0
    e2 = p2 - 2.0 * p1
    e3 = -(3.0 * p2)
    elev = (e0, e1, e2, e3)
    rem = []
    remf = []
    for e in elev:
        v = e / 4.0
        up = _ceil_f32(v) * 4.0
        down = _floor_f32(v) * 4.0
        rf = jnp.where(up - e < e - down, up, down)
        remf.append(rf)
        rem.append(rf.astype(jnp.int32))
    ssum = (rem[0] + rem[1]) + (rem[2] + rem[3])
    s_vec = lax.shift_right_arithmetic(ssum, 2)
    one_i = jnp.full((_LANES,), 1, jnp.int32)
    zero_i = jnp.zeros((_LANES,), jnp.int32)
    d = [e - rf for e, rf in zip(elev, remf)]
    rank = [s_vec, s_vec, s_vec, s_vec]
    for i in range(_DP1):
        for j in range(i + 1, _DP1):
            c = jnp.where(d[j] > d[i], one_i, zero_i)
            rank[i] = rank[i] + c
            rank[j] = rank[j] + (one_i - c)
    for i in range(_DP1):
        adj = (jnp.where(rank[i] < 0, 4, 0) - jnp.where(rank[i] > _POS_DIM, 4, 0)).astype(jnp.int32)
        rank[i] = rank[i] + adj
        rem[i] = rem[i] + adj
    zero_f = jnp.zeros((_LANES,), jnp.float32)
    delta = [(elev[i] - rem[i].astype(jnp.float32)) / 4.0 for i in range(_DP1)]
    g = []
    for j in range(_DP1):
        acc = zero_f
        for i in range(_DP1):
            acc = acc + jnp.where(rank[i] == j, delta[i], zero_f)
        g.append(acc)
    bary0 = g[3] + (jnp.float32(1.0) - g[0])
    bary1 = g[2] - g[3]
    bary2 = g[1] - g[2]
    bary3 = g[0] - g[1]
    u0 = rem[0].astype(jnp.uint32)
    m1 = rem[1].astype(jnp.uint32) * jnp.uint32(_PRIMES[1])
    m2 = rem[2].astype(jnp.uint32) * jnp.uint32(_PRIMES[2])
    mask = jnp.uint32(_HASHMAP_SIZE - 1)
    idxs = []
    for r in range(_DP1):
        if r == 0:
            t0, t1, t2 = u0, m1, m2
        else:
            cr = jnp.uint32((r * _PRIMES[0]) & 0xFFFFFFFF)
            crm4 = jnp.uint32(((r - 4) * _PRIMES[0]) & 0xFFFFFFFF)
            t0 = u0 + jnp.where(rank[0] > _POS_DIM - r, crm4, cr)
            cr = jnp.uint32((r * _PRIMES[1]) & 0xFFFFFFFF)
            crm4 = jnp.uint32(((r - 4) * _PRIMES[1]) & 0xFFFFFFFF)
            t1 = m1 + jnp.where(rank[1] > _POS_DIM - r, crm4, cr)
            cr = jnp.uint32((r * _PRIMES[2]) & 0xFFFFFFFF)
            crm4 = jnp.uint32(((r - 4) * _PRIMES[2]) & 0xFFFFFFFF)
            t2 = m2 + jnp.where(rank[2] > _POS_DIM - r, crm4, cr)
        h = (t0 ^ t1) ^ t2
        idxs.append((h & mask).astype(jnp.int32))
    return idxs, (bary0, bary1, bary2, bary3)


def _sc_body(px_hbm, py_hbm, pz_hbm, tab0_hbm, tab1_hbm, cst_hbm, out_hbm,
             xs, ys, zs, cst_v, idx_v, bary_v, rows0_v, rows1_v,
             ob_v, sem0, sem1):
    c = lax.axis_index("c")
    s = lax.axis_index("s")
    wid = s * _NC + c
    base = wid * _PTS_W
    pltpu.sync_copy(px_hbm.at[pl.ds(base, _PTS_W)], xs)
    pltpu.sync_copy(py_hbm.at[pl.ds(base, _PTS_W)], ys)
    pltpu.sync_copy(pz_hbm.at[pl.ds(base, _PTS_W)], zs)
    pltpu.sync_copy(cst_hbm, cst_v)
    sems = (sem0, sem1)
    iota16 = lax.iota(jnp.int32, _LANES)
    _NF = _N_LEVELS * _N_FEATS

    @pl.loop(0, _NCHUNK)
    def _chunk(ch):
        off = ch * _CHUNK

        def _compute(l):
            so = (l & 1) * (_DP1 * _CHUNK)
            cb = l * (6 * _LANES)
            sc0 = cst_v[pl.ds(cb + 0 * _LANES, _LANES)]
            sc1 = cst_v[pl.ds(cb + 1 * _LANES, _LANES)]
            sc2 = cst_v[pl.ds(cb + 2 * _LANES, _LANES)]
            sh0 = cst_v[pl.ds(cb + 3 * _LANES, _LANES)]
            sh1 = cst_v[pl.ds(cb + 4 * _LANES, _LANES)]
            sh2 = cst_v[pl.ds(cb + 5 * _LANES, _LANES)]
            loff = jnp.full((_LANES,), l * _HASHMAP_SIZE, jnp.int32)

            @pl.loop(0, _NV)
            def _vcompute(v):
                po = off + v * _LANES
                x = xs[pl.ds(po, _LANES)]
                y = ys[pl.ds(po, _LANES)]
                z = zs[pl.ds(po, _LANES)]
                idxs, barys = _vertex_data(x, y, z, sc0, sc1, sc2, sh0, sh1, sh2)
                for r in range(_DP1):
                    idx_v[pl.ds(so + r * _CHUNK + v * _LANES, _LANES)] = idxs[r] + loff
                    bary_v[pl.ds(so + r * _CHUNK + v * _LANES, _LANES)] = barys[r]

        def _fire(l):
            so = (l & 1) * (_DP1 * _CHUNK)
            sem = sems[l & 1]

            @pl.loop(0, _NG)
            def _gfire(g):
                pltpu.async_copy(tab0_hbm.at[idx_v.at[pl.ds(so + g * _GSUB, _GSUB)]],
                                 rows0_v.at[pl.ds(so + g * _GSUB, _GSUB)], sem)
                pltpu.async_copy(tab1_hbm.at[idx_v.at[pl.ds(so + g * _GSUB, _GSUB)]],
                                 rows1_v.at[pl.ds(so + g * _GSUB, _GSUB)], sem)

        def _drain_blend(l):
            so = (l & 1) * (_DP1 * _CHUNK)
            sem = sems[l & 1]
            pltpu.make_async_copy(tab0_hbm.at[pl.ds(0, _DP1 * _CHUNK)],
                                  rows0_v.at[pl.ds(so, _DP1 * _CHUNK)], sem).wait()
            pltpu.make_async_copy(tab1_hbm.at[pl.ds(0, _DP1 * _CHUNK)],
                                  rows1_v.at[pl.ds(so, _DP1 * _CHUNK)], sem).wait()

            @pl.loop(0, _NV)
            def _vblend(v):
                acc0 = jnp.zeros((_LANES,), jnp.float32)
                acc1 = jnp.zeros((_LANES,), jnp.float32)
                for r in range(_DP1):
                    q = so + r * _CHUNK + v * _LANES
                    b = bary_v[pl.ds(q, _LANES)]
                    acc0 = acc0 + b * rows0_v[pl.ds(q, _LANES)]
                    acc1 = acc1 + b * rows1_v[pl.ds(q, _LANES)]
                ob_v[pl.ds(v * _LANES, _LANES)] = acc0
                ob_v[pl.ds(_CHUNK + v * _LANES, _LANES)] = acc1

            obase = (2 * l) * _N_POINTS + base + off
            pltpu.sync_copy(ob_v.at[pl.ds(0, _CHUNK)], out_hbm.at[pl.ds(obase, _CHUNK)])
            pltpu.sync_copy(ob_v.at[pl.ds(_CHUNK, _CHUNK)],
                            out_hbm.at[pl.ds(obase + _N_POINTS, _CHUNK)])

        _compute(0)
        _fire(0)
        for l in range(1, _N_LEVELS):
            _compute(l)
            _fire(l)
            _drain_blend(l - 1)
        _drain_blend(_N_LEVELS - 1)


@jax.jit
def _encode_sc(px, py, pz, tab0, tab1, cst):
    mesh = plsc.VectorSubcoreMesh(core_axis_name="c", subcore_axis_name="s")
    return pl.kernel(
        _sc_body,
        out_type=jax.ShapeDtypeStruct((_N_LEVELS * _N_FEATS * _N_POINTS,), jnp.float32),
        mesh=mesh,
        scratch_types=[
            pltpu.VMEM((_PTS_W,), jnp.float32),
            pltpu.VMEM((_PTS_W,), jnp.float32),
            pltpu.VMEM((_PTS_W,), jnp.float32),
            pltpu.VMEM((_N_LEVELS * 6 * _LANES,), jnp.float32),
            pltpu.VMEM((2 * _DP1 * _CHUNK,), jnp.int32),
            pltpu.VMEM((2 * _DP1 * _CHUNK,), jnp.float32),
            pltpu.VMEM((2 * _DP1 * _CHUNK,), jnp.float32),
            pltpu.VMEM((2 * _DP1 * _CHUNK,), jnp.float32),
            pltpu.VMEM((2 * _CHUNK,), jnp.float32),
            pltpu.SemaphoreType.DMA,
            pltpu.SemaphoreType.DMA,
        ],
    )(px, py, pz, tab0, tab1, cst)


_TBLK = 2048
_PBLK = 2048


def _transpose_body(*refs):
    x = jnp.stack([r[...] for r in refs[:_N_LEVELS * _N_FEATS]], axis=0)
    refs[_N_LEVELS * _N_FEATS][...] = x.T


_FBLK = 8192


def _split_body(a_ref, o0_ref, o1_ref):
    v = a_ref[...]
    o0_ref[...] = v[:, 0]
    o1_ref[...] = v[:, 1]


@jax.jit
def _split_tc(lattice_values):
    nb = _HASHMAP_SIZE // _FBLK
    plane = jax.ShapeDtypeStruct((_N_LEVELS * _HASHMAP_SIZE,), jnp.float32)
    return pl.pallas_call(
        _split_body,
        grid=(_N_LEVELS * nb,),
        in_specs=[pl.BlockSpec((None, _FBLK, _N_FEATS), lambda i: (i // nb, i % nb, 0))],
        out_specs=[pl.BlockSpec((_FBLK,), lambda i: (i,)),
                   pl.BlockSpec((_FBLK,), lambda i: (i,))],
        out_shape=[plane, plane],
    )(lattice_values)


def _prep_body(p_ref, s_ref, x_ref, y_ref, z_ref):
    sc = p_ref[...] * s_ref[...]
    x_ref[...] = sc[:, 0]
    y_ref[...] = sc[:, 1]
    z_ref[...] = sc[:, 2]


@jax.jit
def _prep_tc(positions, pos_scale):
    vec = jax.ShapeDtypeStruct((_N_POINTS,), jnp.float32)
    return pl.pallas_call(
        _prep_body,
        grid=(_N_POINTS // _PBLK,),
        in_specs=[pl.BlockSpec((_PBLK, _POS_DIM), lambda i: (i, 0)),
                  pl.BlockSpec((1, _POS_DIM), lambda i: (0, 0))],
        out_specs=[pl.BlockSpec((_PBLK,), lambda i: (i,))] * 3,
        out_shape=[vec, vec, vec],
    )(positions, pos_scale[None, :])


@jax.jit
def _transpose_tc(y_flat):
    nf = _N_LEVELS * _N_FEATS
    nblk = _N_POINTS // _TBLK

    def _mk(f):
        return pl.BlockSpec((_TBLK,), lambda i, f=f: (f * nblk + i,))

    return pl.pallas_call(
        _transpose_body,
        grid=(nblk,),
        in_specs=[_mk(f) for f in range(nf)],
        out_specs=pl.BlockSpec((_TBLK, nf), lambda i: (i, 0)),
        out_shape=jax.ShapeDtypeStruct((_N_POINTS, nf), jnp.float32),
    )(*([y_flat] * nf))


def kernel(positions, lattice_values, level_random_shifts, pos_scale):
    px, py, pz = _prep_tc(positions, pos_scale)
    tab0, tab1 = _split_tc(lattice_values)
    cst = jnp.zeros((_N_LEVELS, 6), jnp.float32)
    cst = cst.at[:, 0:3].set(jnp.asarray(_SCALE_NP))
    cst = cst.at[:, 3:6].set(level_random_shifts.astype(jnp.float32))
    cst = jnp.broadcast_to(cst[:, :, None], (_N_LEVELS, 6, _LANES)).reshape(-1)
    out = _encode_sc(px, py, pz, tab0, tab1, cst)
    return _transpose_tc(out)
